# baseline (speedup 1.0000x reference)
import functools
import math

import jax
import jax.numpy as jnp
from jax import lax
from jax.experimental import pallas as pl
from jax.experimental.pallas import tpu as pltpu
from jax.experimental.pallas import tpu_sc as plsc

N = 10000
E = 320000
H = 128
G = 100
NL = 6
DIST_DIM = 12
W0 = 144
NW = 32
PER_W = E // NW
NB = 3
EB = 512
NPAD = 10240
RT = NPAD // 16

_FREQS = [2.0 * math.pi * (4.0 ** k) / 15.0 for k in range(DIST_DIM // 2)]


def _silu(v):
    return v * jax.nn.sigmoid(v)


def _unpack_pairs(x):
    u = lax.bitcast_convert_type(x, jnp.uint32)
    lo = lax.bitcast_convert_type(u << 16, jnp.float32)
    hi = lax.bitcast_convert_type(u & jnp.uint32(0xFFFF0000), jnp.float32)
    return jnp.concatenate([lo, hi], axis=1)


def _make_sc_gather(with_radial):
    mesh = plsc.VectorSubcoreMesh(core_axis_name="c", subcore_axis_name="s")
    f32 = jnp.float32
    i32 = jnp.int32
    CK = 128
    NF = PER_W // CK
    NF -= NF % NB
    TL = PER_W - NF * CK

    WP = H // 2
    out_type = [jax.ShapeDtypeStruct((E, WP), f32),
                jax.ShapeDtypeStruct((E, WP), f32)]
    scratch = []
    for _ in range(NB):
        scratch.extend([
            pltpu.VMEM((CK,), i32), pltpu.VMEM((CK,), i32),
            pltpu.VMEM((CK, WP), f32), pltpu.VMEM((CK, WP), f32),
            pltpu.SemaphoreType.DMA, pltpu.SemaphoreType.DMA,
            pltpu.SemaphoreType.DMA,
        ])
    scratch.extend([
        pltpu.VMEM((TL,), i32), pltpu.VMEM((TL,), i32),
        pltpu.VMEM((TL, WP), f32), pltpu.VMEM((TL, WP), f32),
    ])
    if with_radial:
        out_type.append(jax.ShapeDtypeStruct((E,), f32))
        scratch.append(pltpu.VMEM((N * 4,), f32))
        for _ in range(NB):
            scratch.append(pltpu.VMEM((CK,), f32))
        scratch.append(pltpu.VMEM((TL,), f32))

    @functools.partial(
        pl.kernel,
        out_type=tuple(out_type),
        mesh=mesh,
        scratch_types=scratch,
        compiler_params=pltpu.CompilerParams(
            needs_layout_passes=False, use_tc_tiling_on_sc=False),
    )
    def gather_k(*refs):
        atab, btab, row, col = refs[:4]
        k = 4
        if with_radial:
            pos4 = refs[k]; k += 1
        oa, ob = refs[k:k + 2]; k += 2
        if with_radial:
            orad = refs[k]; k += 1
        ridx, cidx, bufa, bufb, isem, gsem, wsem = [], [], [], [], [], [], []
        for _ in range(NB):
            ridx.append(refs[k]); cidx.append(refs[k + 1])
            bufa.append(refs[k + 2]); bufb.append(refs[k + 3])
            isem.append(refs[k + 4]); gsem.append(refs[k + 5])
            wsem.append(refs[k + 6])
            k += 7
        ridxt, cidxt, bufat, bufbt = refs[k:k + 4]; k += 4
        if with_radial:
            posv = refs[k]; k += 1
            radb = refs[k:k + NB]; k += NB
            radbt = refs[k]; k += 1

        wid = lax.axis_index("s") * 2 + lax.axis_index("c")
        base = wid * PER_W
        if with_radial:
            pltpu.sync_copy(pos4, posv)

        def radial_into(rref, cref, dst, n):
            for g in range(n // 16):
                r16 = rref[pl.ds(g * 16, 16)] * 4
                c16 = cref[pl.ds(g * 16, 16)] * 4
                acc = jnp.zeros((16,), f32)
                for comp in range(3):
                    dv = (plsc.load_gather(posv, [r16 + comp])
                          - plsc.load_gather(posv, [c16 + comp]))
                    acc = acc + dv * dv
                dst[pl.ds(g * 16, 16)] = acc

        def issue_idx(c, b):
            off = base + c * CK
            pltpu.async_copy(row.at[pl.ds(off, CK)], ridx[b], isem[b])
            pltpu.async_copy(col.at[pl.ds(off, CK)], cidx[b], isem[b])

        def take_gather(c, b):
            pltpu.make_async_copy(row.at[pl.ds(base + c * CK, CK)],
                                  ridx[b], isem[b]).wait()
            pltpu.make_async_copy(col.at[pl.ds(base + c * CK, CK)],
                                  cidx[b], isem[b]).wait()
            pltpu.async_copy(atab.at[ridx[b]], bufa[b], gsem[b])
            pltpu.async_copy(btab.at[cidx[b]], bufb[b], gsem[b])
            if with_radial:
                radial_into(ridx[b], cidx[b], radb[b], CK)

        def issue_write(c, b):
            off = base + c * CK
            pltpu.make_async_copy(atab.at[ridx[b]], bufa[b], gsem[b]).wait()
            pltpu.make_async_copy(btab.at[cidx[b]], bufb[b], gsem[b]).wait()
            pltpu.async_copy(bufa[b], oa.at[pl.ds(off, CK)], wsem[b])
            pltpu.async_copy(bufb[b], ob.at[pl.ds(off, CK)], wsem[b])
            if with_radial:
                pltpu.async_copy(radb[b], orad.at[pl.ds(off, CK)], wsem[b])

        def wait_write(c, b):
            off = base + c * CK
            pltpu.make_async_copy(bufa[b], oa.at[pl.ds(off, CK)],
                                  wsem[b]).wait()
            pltpu.make_async_copy(bufb[b], ob.at[pl.ds(off, CK)],
                                  wsem[b]).wait()
            if with_radial:
                pltpu.make_async_copy(radb[b], orad.at[pl.ds(off, CK)],
                                      wsem[b]).wait()

        def body(j, carry):
            for b in range(NB):
                i = j * NB + b

                @pl.when(j >= 1)
                def _(b=b, i=i):
                    wait_write(i - NB, b)

                issue_idx(i, b)
                if b == 0:
                    @pl.when(j >= 1)
                    def _(b=b, i=i):
                        take_gather(i - 1, (b - 1) % NB)
                else:
                    take_gather(i - 1, b - 1)
                if b <= 1:
                    @pl.when(j >= 1)
                    def _(b=b, i=i):
                        issue_write(i - 2, (b - 2) % NB)
                else:
                    issue_write(i - 2, b - 2)
            return carry

        lax.fori_loop(0, NF // NB, body, 0)

        L = NF - 1
        take_gather(L, L % NB)
        issue_write(L - 1, (L - 1) % NB)
        issue_write(L, L % NB)
        wait_write(L - 2, (L - 2) % NB)
        wait_write(L - 1, (L - 1) % NB)
        wait_write(L, L % NB)

        off = base + NF * CK
        pltpu.sync_copy(row.at[pl.ds(off, TL)], ridxt)
        pltpu.sync_copy(col.at[pl.ds(off, TL)], cidxt)
        ca = pltpu.async_copy(atab.at[ridxt], bufat, gsem[0])
        cb = pltpu.async_copy(btab.at[cidxt], bufbt, gsem[1])
        if with_radial:
            radial_into(ridxt, cidxt, radbt, TL)
        ca.wait()
        cb.wait()
        pltpu.sync_copy(bufat, oa.at[pl.ds(off, TL)])
        pltpu.sync_copy(bufbt, ob.at[pl.ds(off, TL)])
        if with_radial:
            pltpu.sync_copy(radbt, orad.at[pl.ds(off, TL)])

    return gather_k


_make_sc_gather = functools.lru_cache(maxsize=None)(_make_sc_gather)


def _make_sc_scatter():
    mesh = plsc.VectorSubcoreMesh(core_axis_name="c", subcore_axis_name="s")
    f32 = jnp.float32
    HC = H // 2
    SPER = E // 16
    CK = 128
    SNFULL = SPER // CK
    SNFULL -= SNFULL % NB
    STAIL = SPER - SNFULL * CK

    scratch = [pltpu.VMEM((RT, HC), f32)]
    for _ in range(NB):
        scratch.extend([
            pltpu.VMEM((CK,), jnp.int32), pltpu.VMEM((CK, HC), f32),
            pltpu.SemaphoreType.DMA, pltpu.SemaphoreType.DMA,
        ])
    scratch.extend([
        pltpu.VMEM((STAIL,), jnp.int32), pltpu.VMEM((STAIL, HC), f32),
        pltpu.VMEM_SHARED((NPAD, HC), f32),
    ])

    @functools.partial(
        pl.kernel,
        out_type=jax.ShapeDtypeStruct((NPAD, H), f32),
        mesh=mesh,
        scratch_types=scratch,
        compiler_params=pltpu.CompilerParams(
            needs_layout_passes=False, use_tc_tiling_on_sc=False),
    )
    def scatter_k(*refs):
        m, row, zeros_h, out = refs[:4]
        stage = refs[4]
        k = 5
        idxb, mbuf, lsem, ssem = [], [], [], []
        for _ in range(NB):
            idxb.append(refs[k]); mbuf.append(refs[k + 1])
            lsem.append(refs[k + 2]); ssem.append(refs[k + 3])
            k += 4
        idxt, mbuft, shared = refs[k:k + 3]

        c = lax.axis_index("c")
        s = lax.axis_index("s")
        base = s * SPER
        col0 = c * HC

        pltpu.sync_copy(zeros_h.at[pl.ds(s * RT, RT)], stage)
        pltpu.sync_copy(stage, shared.at[pl.ds(s * RT, RT)])
        plsc.subcore_barrier()

        def issue_load(i, b):
            off = base + i * CK
            pltpu.async_copy(row.at[pl.ds(off, CK)], idxb[b], lsem[b])
            pltpu.async_copy(m.at[pl.ds(off, CK), pl.ds(col0, HC)],
                             mbuf[b], lsem[b])

        def take_scatter(i, b):
            off = base + i * CK
            pltpu.make_async_copy(row.at[pl.ds(off, CK)], idxb[b],
                                  lsem[b]).wait()
            pltpu.make_async_copy(m.at[pl.ds(off, CK), pl.ds(col0, HC)],
                                  mbuf[b], lsem[b]).wait()
            pltpu.async_copy(mbuf[b], shared.at[idxb[b]], ssem[b], add=True)

        def wait_scatter(b):
            pltpu.make_async_copy(mbuf[b], shared.at[idxb[b]],
                                  ssem[b]).wait()

        def body(j, carry):
            for b in range(NB):
                i = j * NB + b

                @pl.when(j >= 1)
                def _(b=b):
                    wait_scatter(b)

                issue_load(i, b)
                if b == 0:
                    @pl.when(j >= 1)
                    def _(b=b, i=i):
                        take_scatter(i - 1, (b - 1) % NB)
                else:
                    take_scatter(i - 1, b - 1)
            return carry

        lax.fori_loop(0, SNFULL // NB, body, 0)

        L = SNFULL - 1
        take_scatter(L, L % NB)
        for b in range(NB):
            wait_scatter(b)

        if STAIL:
            off = base + SNFULL * CK
            pltpu.sync_copy(row.at[pl.ds(off, STAIL)], idxt)
            pltpu.sync_copy(m.at[pl.ds(off, STAIL), pl.ds(col0, HC)], mbuft)
            pltpu.sync_copy(mbuft, shared.at[idxt], add=True)

        plsc.subcore_barrier()
        pltpu.sync_copy(shared.at[pl.ds(s * RT, RT)], stage)
        pltpu.sync_copy(stage, out.at[pl.ds(s * RT, RT), pl.ds(col0, HC)])

    return scatter_k


_make_sc_scatter = functools.lru_cache(maxsize=None)(_make_sc_scatter)


def _gather_tables(atab, btab, row, col):
    return _make_sc_gather(False)(atab, btab, row, col)


def _gather_tables_rad(atab, btab, row, col, pos4):
    return _make_sc_gather(True)(atab, btab, row, col, pos4)


def _scatter_sum(m, row, zeros_h):
    return _make_sc_scatter()(m, row, zeros_h)


def _full(shape):
    return pl.BlockSpec(shape, lambda: tuple(0 for _ in shape))


def _prep_call(x, pring, Win, b_in, ring0, ring1, wea0, web0, be10):
    f32 = jnp.float32

    def body(x_r, pr_r, win_r, bin_r, r0_r, r1_r, wa_r, wb_r, be_r,
             h_o, a_o, b_o):
        p = pr_r[...]
        h0 = (jnp.dot(x_r[...], win_r[...], preferred_element_type=f32)
              + bin_r[...] + (1.0 - p) * r0_r[...] + p * r1_r[...])
        h_o[...] = h0
        a_o[...] = (jnp.dot(h0, wa_r[...], preferred_element_type=f32)
                    + be_r[...]).astype(jnp.bfloat16)
        b_o[...] = jnp.dot(h0, wb_r[...],
                           preferred_element_type=f32).astype(jnp.bfloat16)

    return pl.pallas_call(
        body,
        out_shape=(jax.ShapeDtypeStruct((N, H), f32),
                   jax.ShapeDtypeStruct((N, H), jnp.bfloat16),
                   jax.ShapeDtypeStruct((N, H), jnp.bfloat16)),
    )(x, pring, Win, b_in, ring0, ring1, wea0, web0, be10)


def _edge0_call(ga, gb, rad2d, edge_attr, Wb1, bb1, Wb2, bb2, w1t, we2, be2,
                wattr, battb):
    f32 = jnp.float32

    def body(ga_r, gb_r, rad_r, ea_r, wb1_r, bb1_r, wb2_r, bb2_r, w1t_r, we2_r,
             be2_r, watt_r, batt_r, eat_o, m_o):
        a = _unpack_pairs(ga_r[...]) + _unpack_pairs(gb_r[...])
        radial = rad_r[...]
        d = jnp.sqrt(radial + 1e-8)
        kidx = lax.broadcasted_iota(jnp.int32, (1, DIST_DIM // 2), 1)
        freqs = (2.0 * math.pi / 15.0) * jnp.exp2(2.0 * kidx.astype(f32))
        ang = d * freqs
        bond = jnp.dot(_silu(jnp.dot(ea_r[...], wb1_r[...],
                                     preferred_element_type=f32) + bb1_r[...]),
                       wb2_r[...], preferred_element_type=f32) + bb2_r[...]
        eat = jnp.concatenate(
            [jnp.sin(ang), jnp.cos(ang), bond, jnp.zeros((EB, 4), f32)], axis=1)
        eat_o[...] = eat
        pre = a + jnp.dot(eat, w1t_r[...], preferred_element_type=f32)
        q = _silu(jnp.dot(_silu(pre), we2_r[...], preferred_element_type=f32)
                  + be2_r[...])
        alog = (jnp.sum(q * watt_r[...], axis=1, keepdims=True)
                + batt_r[...][:, :1])
        m_o[...] = q * jax.nn.sigmoid(alog)

    grid = (E // EB,)
    return pl.pallas_call(
        body,
        grid=grid,
        in_specs=[
            pl.BlockSpec((EB, H // 2), lambda i: (i, 0)),
            pl.BlockSpec((EB, H // 2), lambda i: (i, 0)),
            pl.BlockSpec((EB, 1), lambda i: (i, 0)),
            pl.BlockSpec((EB, 16), lambda i: (i, 0)),
            pl.BlockSpec((16, 16), lambda i: (0, 0)),
            pl.BlockSpec((1, 16), lambda i: (0, 0)),
            pl.BlockSpec((16, 16), lambda i: (0, 0)),
            pl.BlockSpec((1, 16), lambda i: (0, 0)),
            pl.BlockSpec((32, H), lambda i: (0, 0)),
            pl.BlockSpec((H, H), lambda i: (0, 0)),
            pl.BlockSpec((1, H), lambda i: (0, 0)),
            pl.BlockSpec((1, H), lambda i: (0, 0)),
            pl.BlockSpec((1, H), lambda i: (0, 0)),
        ],
        out_specs=(pl.BlockSpec((EB, 32), lambda i: (i, 0)),
                   pl.BlockSpec((EB, H), lambda i: (i, 0))),
        out_shape=(jax.ShapeDtypeStruct((E, 32), f32),
                   jax.ShapeDtypeStruct((E, H), f32)),
    )(ga, gb, rad2d, edge_attr, Wb1, bb1, Wb2, bb2, w1t, we2, be2, wattr, battb)


def _edge_call(ga, gb, eat, w1t, we2, be2, wattr, battb):
    f32 = jnp.float32

    def body(ga_r, gb_r, ea_r, w1t_r, we2_r, be2_r, watt_r, batt_r, m_o):
        pre = (_unpack_pairs(ga_r[...]) + _unpack_pairs(gb_r[...])
               + jnp.dot(ea_r[...], w1t_r[...], preferred_element_type=f32))
        q = _silu(jnp.dot(_silu(pre), we2_r[...], preferred_element_type=f32)
                  + be2_r[...])
        alog = (jnp.sum(q * watt_r[...], axis=1, keepdims=True)
                + batt_r[...][:, :1])
        m_o[...] = q * jax.nn.sigmoid(alog)

    grid = (E // EB,)
    return pl.pallas_call(
        body,
        grid=grid,
        in_specs=[
            pl.BlockSpec((EB, H // 2), lambda i: (i, 0)),
            pl.BlockSpec((EB, H // 2), lambda i: (i, 0)),
            pl.BlockSpec((EB, 32), lambda i: (i, 0)),
            pl.BlockSpec((32, H), lambda i: (0, 0)),
            pl.BlockSpec((H, H), lambda i: (0, 0)),
            pl.BlockSpec((1, H), lambda i: (0, 0)),
            pl.BlockSpec((1, H), lambda i: (0, 0)),
            pl.BlockSpec((1, H), lambda i: (0, 0)),
        ],
        out_specs=pl.BlockSpec((EB, H), lambda i: (i, 0)),
        out_shape=jax.ShapeDtypeStruct((E, H), f32),
    )(ga, gb, eat, w1t, we2, be2, wattr, battb)


def _node_call(h, parts, wn1a, wn1b, bn1, wn2, bn2, wea, web, be1n):
    f32 = jnp.float32

    def body(h_r, p_r, wa_r, wb_r, b1_r, w2_r, b2_r, wea_r, web_r, be_r,
             h_o, a_o, b_o):
        h0 = h_r[...]
        agg = p_r[...][:N, :]
        t = _silu(jnp.dot(h0, wa_r[...], preferred_element_type=f32)
                  + jnp.dot(agg, wb_r[...], preferred_element_type=f32)
                  + b1_r[...])
        hn = h0 + jnp.dot(t, w2_r[...], preferred_element_type=f32) + b2_r[...]
        h_o[...] = hn
        a_o[...] = (jnp.dot(hn, wea_r[...], preferred_element_type=f32)
                    + be_r[...]).astype(jnp.bfloat16)
        b_o[...] = jnp.dot(hn, web_r[...],
                           preferred_element_type=f32).astype(jnp.bfloat16)

    return pl.pallas_call(
        body,
        out_shape=(jax.ShapeDtypeStruct((N, H), f32),
                   jax.ShapeDtypeStruct((N, H), jnp.bfloat16),
                   jax.ShapeDtypeStruct((N, H), jnp.bfloat16)),
    )(h, parts, wn1a, wn1b, bn1, wn2, bn2, wea, web, be1n)


def _final_call(h, parts, wn1a, wn1b, bn1, wn2, bn2, Wo1, bo1, wo2r, bo2b,
                batch2d):
    f32 = jnp.float32

    def body(h_r, p_r, wa_r, wb_r, b1_r, w2_r, b2_r, wo1_r, bo1_r, wo2_r,
             bo2_r, bat_r, out_o):
        h0 = h_r[...]
        agg = p_r[...][:N, :]
        t = _silu(jnp.dot(h0, wa_r[...], preferred_element_type=f32)
                  + jnp.dot(agg, wb_r[...], preferred_element_type=f32)
                  + b1_r[...])
        hn = h0 + jnp.dot(t, w2_r[...], preferred_element_type=f32) + b2_r[...]
        u = jax.nn.relu(jnp.dot(hn, wo1_r[...], preferred_element_type=f32)
                        + bo1_r[...])
        logits = (jnp.sum(u * wo2_r[...], axis=1, keepdims=True)
                  + bo2_r[...][:, :1])
        gids = lax.broadcasted_iota(jnp.int32, (1, G), 1)
        mask = bat_r[...] == gids
        neg = jnp.float32(-1e30)
        cnt = jnp.sum(mask.astype(f32), axis=0, keepdims=True)
        gmax = jnp.max(jnp.where(mask, logits, neg), axis=0, keepdims=True)
        gmax = jnp.where(cnt > 0.0, gmax, 0.0)
        gmax_n = jnp.sum(jnp.where(mask, gmax, 0.0), axis=1, keepdims=True)
        ex = jnp.exp(logits - gmax_n)
        z = jnp.sum(jnp.where(mask, ex, 0.0), axis=0, keepdims=True)
        z_n = jnp.sum(jnp.where(mask, z, 0.0), axis=1, keepdims=True)
        probs = ex / (z_n + 1e-12)
        pmax = jnp.max(jnp.where(mask, probs, neg), axis=0, keepdims=True)
        pmax = jnp.where(cnt > 0.0, pmax, 0.0)
        out_o[...] = jnp.log(pmax + 1e-9)

    return pl.pallas_call(
        body,
        out_shape=jax.ShapeDtypeStruct((1, G), f32),
    )(h, parts, wn1a, wn1b, bn1, wn2, bn2, Wo1, bo1, wo2r, bo2b, batch2d)


def kernel(x, pos, edge_index, edge_attr, pring_out, batch,
           Wb1, bb1, Wb2, bb2, Win, b_in, ring_emb,
           We1, be1, We2, be2, Wn1, bn1, Wn2, bn2, Watt, batt,
           Wo1, bo1, Wo2, bo2):
    f32 = jnp.float32
    row = edge_index[0]
    col = edge_index[1]

    perm = jnp.concatenate([jnp.arange(0, H, 2), jnp.arange(1, H, 2)])
    wea = We1[:, :H, :]
    web = We1[:, H:2 * H, :]
    w1t = jnp.pad(We1[:, 2 * H:, :], ((0, 0), (0, 4), (0, 0)))[:, :, perm]
    we2p = We2[:, perm, :]
    be1r = be1.reshape(NL, 1, H)
    be2r = be2.reshape(NL, 1, H)
    bn1r = bn1.reshape(NL, 1, H)
    bn2r = bn2.reshape(NL, 1, H)
    wn1a = Wn1[:, :H, :]
    wn1b = Wn1[:, H:, :]
    wattr = Watt[:, :, 0].reshape(NL, 1, H)
    battb = jnp.broadcast_to(batt.reshape(NL, 1, 1), (NL, 1, H))
    bb1r = bb1.reshape(1, 16)
    bb2r = bb2.reshape(1, 16)
    b_inr = b_in.reshape(1, H)
    ring0 = ring_emb[0:1, :]
    ring1 = ring_emb[1:2, :]
    bo1r = bo1.reshape(1, 2 * H)
    wo2r = Wo2.reshape(1, 2 * H)
    bo2b = jnp.broadcast_to(bo2.reshape(1, 1), (1, H))
    pring = pring_out.astype(f32).reshape(N, 1)
    pos4 = jnp.pad(pos, ((0, 0), (0, 1)))
    batch2d = batch.reshape(N, 1)
    zeros_h = jnp.zeros((NPAD, H // 2), f32)

    def pack(t):
        return lax.bitcast_convert_type(t.reshape(N, H // 2, 2), f32)

    h, abf, bbf = _prep_call(x, pring, Win, b_inr, ring0, ring1,
                             wea[0], web[0], be1r[0])
    atab, btab = pack(abf), pack(bbf)

    eat = None
    out = None
    for l in range(NL):
        if l == 0:
            ga, gb, rad = _gather_tables_rad(atab, btab, row, col,
                                             pos4.reshape(N * 4))
            eat, m = _edge0_call(ga, gb, rad.reshape(E, 1), edge_attr,
                                 Wb1, bb1r, Wb2, bb2r,
                                 w1t[0], we2p[0], be2r[0], wattr[0], battb[0])
        else:
            ga, gb = _gather_tables(atab, btab, row, col)
            m = _edge_call(ga, gb, eat, w1t[l], we2p[l], be2r[l],
                           wattr[l], battb[l])
        parts = _scatter_sum(m, row, zeros_h)
        if l < NL - 1:
            h, abf, bbf = _node_call(h, parts, wn1a[l], wn1b[l], bn1r[l],
                                     Wn2[l], bn2r[l], wea[l + 1],
                                     web[l + 1], be1r[l + 1])
            atab, btab = pack(abf), pack(bbf)
        else:
            out = _final_call(h, parts, wn1a[l], wn1b[l], bn1r[l], Wn2[l],
                              bn2r[l], Wo1, bo1r, wo2r, bo2b, batch2d)

    return out.reshape(G)


# speedup vs baseline: 1.0267x; 1.0267x over previous
import functools
import math

import jax
import jax.numpy as jnp
from jax import lax
from jax.experimental import pallas as pl
from jax.experimental.pallas import tpu as pltpu
from jax.experimental.pallas import tpu_sc as plsc

N = 10000
E = 320000
H = 128
G = 100
NL = 6
DIST_DIM = 12
W0 = 144
NW = 32
PER_W = E // NW
NB = 3
EB = 512
NPAD = 10240
RT = NPAD // 16

_FREQS = [2.0 * math.pi * (4.0 ** k) / 15.0 for k in range(DIST_DIM // 2)]


def _silu(v):
    return v * jax.nn.sigmoid(v)


def _unpack_halves(x):
    u = lax.bitcast_convert_type(x, jnp.uint32)
    lo = lax.bitcast_convert_type(u << 16, jnp.float32)
    hi = lax.bitcast_convert_type(u & jnp.uint32(0xFFFF0000), jnp.float32)
    return lo, hi


def _pack_halves(a):
    u16 = jnp.uint16
    u32 = jnp.uint32
    lo = lax.bitcast_convert_type(a[:, :64].astype(jnp.bfloat16), u16)
    hi = lax.bitcast_convert_type(a[:, 64:].astype(jnp.bfloat16), u16)
    packed = lo.astype(u32) | (hi.astype(u32) << 16)
    return lax.bitcast_convert_type(packed, jnp.float32)


def _make_sc_gather(with_radial):
    mesh = plsc.VectorSubcoreMesh(core_axis_name="c", subcore_axis_name="s")
    f32 = jnp.float32
    i32 = jnp.int32
    CK = 128
    NF = PER_W // CK
    NF -= NF % NB
    TL = PER_W - NF * CK

    WP = H // 2
    out_type = [jax.ShapeDtypeStruct((E, WP), f32),
                jax.ShapeDtypeStruct((E, WP), f32)]
    scratch = []
    for _ in range(NB):
        scratch.extend([
            pltpu.VMEM((CK,), i32), pltpu.VMEM((CK,), i32),
            pltpu.VMEM((CK, WP), f32), pltpu.VMEM((CK, WP), f32),
            pltpu.SemaphoreType.DMA, pltpu.SemaphoreType.DMA,
            pltpu.SemaphoreType.DMA,
        ])
    scratch.extend([
        pltpu.VMEM((TL,), i32), pltpu.VMEM((TL,), i32),
        pltpu.VMEM((TL, WP), f32), pltpu.VMEM((TL, WP), f32),
    ])
    if with_radial:
        out_type.append(jax.ShapeDtypeStruct((E,), f32))
        scratch.append(pltpu.VMEM((N * 4,), f32))
        for _ in range(NB):
            scratch.append(pltpu.VMEM((CK,), f32))
        scratch.append(pltpu.VMEM((TL,), f32))

    @functools.partial(
        pl.kernel,
        out_type=tuple(out_type),
        mesh=mesh,
        scratch_types=scratch,
        compiler_params=pltpu.CompilerParams(
            needs_layout_passes=False, use_tc_tiling_on_sc=False),
    )
    def gather_k(*refs):
        atab, btab, row, col = refs[:4]
        k = 4
        if with_radial:
            pos4 = refs[k]; k += 1
        oa, ob = refs[k:k + 2]; k += 2
        if with_radial:
            orad = refs[k]; k += 1
        ridx, cidx, bufa, bufb, isem, gsem, wsem = [], [], [], [], [], [], []
        for _ in range(NB):
            ridx.append(refs[k]); cidx.append(refs[k + 1])
            bufa.append(refs[k + 2]); bufb.append(refs[k + 3])
            isem.append(refs[k + 4]); gsem.append(refs[k + 5])
            wsem.append(refs[k + 6])
            k += 7
        ridxt, cidxt, bufat, bufbt = refs[k:k + 4]; k += 4
        if with_radial:
            posv = refs[k]; k += 1
            radb = refs[k:k + NB]; k += NB
            radbt = refs[k]; k += 1

        wid = lax.axis_index("s") * 2 + lax.axis_index("c")
        base = wid * PER_W
        if with_radial:
            pltpu.sync_copy(pos4, posv)

        def radial_into(rref, cref, dst, n):
            for g in range(n // 16):
                r16 = rref[pl.ds(g * 16, 16)] * 4
                c16 = cref[pl.ds(g * 16, 16)] * 4
                acc = jnp.zeros((16,), f32)
                for comp in range(3):
                    dv = (plsc.load_gather(posv, [r16 + comp])
                          - plsc.load_gather(posv, [c16 + comp]))
                    acc = acc + dv * dv
                dst[pl.ds(g * 16, 16)] = acc

        def issue_idx(c, b):
            off = base + c * CK
            pltpu.async_copy(row.at[pl.ds(off, CK)], ridx[b], isem[b])
            pltpu.async_copy(col.at[pl.ds(off, CK)], cidx[b], isem[b])

        def take_gather(c, b):
            pltpu.make_async_copy(row.at[pl.ds(base + c * CK, CK)],
                                  ridx[b], isem[b]).wait()
            pltpu.make_async_copy(col.at[pl.ds(base + c * CK, CK)],
                                  cidx[b], isem[b]).wait()
            pltpu.async_copy(atab.at[ridx[b]], bufa[b], gsem[b])
            pltpu.async_copy(btab.at[cidx[b]], bufb[b], gsem[b])
            if with_radial:
                radial_into(ridx[b], cidx[b], radb[b], CK)

        def issue_write(c, b):
            off = base + c * CK
            pltpu.make_async_copy(atab.at[ridx[b]], bufa[b], gsem[b]).wait()
            pltpu.make_async_copy(btab.at[cidx[b]], bufb[b], gsem[b]).wait()
            pltpu.async_copy(bufa[b], oa.at[pl.ds(off, CK)], wsem[b])
            pltpu.async_copy(bufb[b], ob.at[pl.ds(off, CK)], wsem[b])
            if with_radial:
                pltpu.async_copy(radb[b], orad.at[pl.ds(off, CK)], wsem[b])

        def wait_write(c, b):
            off = base + c * CK
            pltpu.make_async_copy(bufa[b], oa.at[pl.ds(off, CK)],
                                  wsem[b]).wait()
            pltpu.make_async_copy(bufb[b], ob.at[pl.ds(off, CK)],
                                  wsem[b]).wait()
            if with_radial:
                pltpu.make_async_copy(radb[b], orad.at[pl.ds(off, CK)],
                                      wsem[b]).wait()

        def body(j, carry):
            for b in range(NB):
                i = j * NB + b

                @pl.when(j >= 1)
                def _(b=b, i=i):
                    wait_write(i - NB, b)

                issue_idx(i, b)
                if b == 0:
                    @pl.when(j >= 1)
                    def _(b=b, i=i):
                        take_gather(i - 1, (b - 1) % NB)
                else:
                    take_gather(i - 1, b - 1)
                if b <= 1:
                    @pl.when(j >= 1)
                    def _(b=b, i=i):
                        issue_write(i - 2, (b - 2) % NB)
                else:
                    issue_write(i - 2, b - 2)
            return carry

        lax.fori_loop(0, NF // NB, body, 0)

        L = NF - 1
        take_gather(L, L % NB)
        issue_write(L - 1, (L - 1) % NB)
        issue_write(L, L % NB)
        wait_write(L - 2, (L - 2) % NB)
        wait_write(L - 1, (L - 1) % NB)
        wait_write(L, L % NB)

        off = base + NF * CK
        pltpu.sync_copy(row.at[pl.ds(off, TL)], ridxt)
        pltpu.sync_copy(col.at[pl.ds(off, TL)], cidxt)
        ca = pltpu.async_copy(atab.at[ridxt], bufat, gsem[0])
        cb = pltpu.async_copy(btab.at[cidxt], bufbt, gsem[1])
        if with_radial:
            radial_into(ridxt, cidxt, radbt, TL)
        ca.wait()
        cb.wait()
        pltpu.sync_copy(bufat, oa.at[pl.ds(off, TL)])
        pltpu.sync_copy(bufbt, ob.at[pl.ds(off, TL)])
        if with_radial:
            pltpu.sync_copy(radbt, orad.at[pl.ds(off, TL)])

    return gather_k


_make_sc_gather = functools.lru_cache(maxsize=None)(_make_sc_gather)


def _make_sc_scatter():
    mesh = plsc.VectorSubcoreMesh(core_axis_name="c", subcore_axis_name="s")
    f32 = jnp.float32
    HC = H // 2
    SPER = E // 16
    CK = 128
    SNFULL = SPER // CK
    SNFULL -= SNFULL % NB
    STAIL = SPER - SNFULL * CK

    scratch = [pltpu.VMEM((RT, HC), f32)]
    for _ in range(NB):
        scratch.extend([
            pltpu.VMEM((CK,), jnp.int32), pltpu.VMEM((CK, HC), f32),
            pltpu.SemaphoreType.DMA, pltpu.SemaphoreType.DMA,
        ])
    scratch.extend([
        pltpu.VMEM((STAIL,), jnp.int32), pltpu.VMEM((STAIL, HC), f32),
        pltpu.VMEM_SHARED((NPAD, HC), f32),
    ])

    @functools.partial(
        pl.kernel,
        out_type=jax.ShapeDtypeStruct((NPAD, H), f32),
        mesh=mesh,
        scratch_types=scratch,
        compiler_params=pltpu.CompilerParams(
            needs_layout_passes=False, use_tc_tiling_on_sc=False),
    )
    def scatter_k(*refs):
        m, row, zeros_h, out = refs[:4]
        stage = refs[4]
        k = 5
        idxb, mbuf, lsem, ssem = [], [], [], []
        for _ in range(NB):
            idxb.append(refs[k]); mbuf.append(refs[k + 1])
            lsem.append(refs[k + 2]); ssem.append(refs[k + 3])
            k += 4
        idxt, mbuft, shared = refs[k:k + 3]

        c = lax.axis_index("c")
        s = lax.axis_index("s")
        base = s * SPER
        col0 = c * HC

        pltpu.sync_copy(zeros_h.at[pl.ds(s * RT, RT)], stage)
        pltpu.sync_copy(stage, shared.at[pl.ds(s * RT, RT)])
        plsc.subcore_barrier()

        def issue_load(i, b):
            off = base + i * CK
            pltpu.async_copy(row.at[pl.ds(off, CK)], idxb[b], lsem[b])
            pltpu.async_copy(m.at[pl.ds(off, CK), pl.ds(col0, HC)],
                             mbuf[b], lsem[b])

        def take_scatter(i, b):
            off = base + i * CK
            pltpu.make_async_copy(row.at[pl.ds(off, CK)], idxb[b],
                                  lsem[b]).wait()
            pltpu.make_async_copy(m.at[pl.ds(off, CK), pl.ds(col0, HC)],
                                  mbuf[b], lsem[b]).wait()
            pltpu.async_copy(mbuf[b], shared.at[idxb[b]], ssem[b], add=True)

        def wait_scatter(b):
            pltpu.make_async_copy(mbuf[b], shared.at[idxb[b]],
                                  ssem[b]).wait()

        def body(j, carry):
            for b in range(NB):
                i = j * NB + b

                @pl.when(j >= 1)
                def _(b=b):
                    wait_scatter(b)

                issue_load(i, b)
                if b == 0:
                    @pl.when(j >= 1)
                    def _(b=b, i=i):
                        take_scatter(i - 1, (b - 1) % NB)
                else:
                    take_scatter(i - 1, b - 1)
            return carry

        lax.fori_loop(0, SNFULL // NB, body, 0)

        L = SNFULL - 1
        take_scatter(L, L % NB)
        for b in range(NB):
            wait_scatter(b)

        if STAIL:
            off = base + SNFULL * CK
            pltpu.sync_copy(row.at[pl.ds(off, STAIL)], idxt)
            pltpu.sync_copy(m.at[pl.ds(off, STAIL), pl.ds(col0, HC)], mbuft)
            pltpu.sync_copy(mbuft, shared.at[idxt], add=True)

        plsc.subcore_barrier()
        pltpu.sync_copy(shared.at[pl.ds(s * RT, RT)], stage)
        pltpu.sync_copy(stage, out.at[pl.ds(s * RT, RT), pl.ds(col0, HC)])

    return scatter_k


_make_sc_scatter = functools.lru_cache(maxsize=None)(_make_sc_scatter)


def _gather_tables(atab, btab, row, col):
    return _make_sc_gather(False)(atab, btab, row, col)


def _gather_tables_rad(atab, btab, row, col, pos4):
    return _make_sc_gather(True)(atab, btab, row, col, pos4)


def _scatter_sum(m, row, zeros_h):
    return _make_sc_scatter()(m, row, zeros_h)


def _full(shape):
    return pl.BlockSpec(shape, lambda: tuple(0 for _ in shape))


def _prep_call(x, pring, Win, b_in, ring0, ring1, wea0, web0, be10):
    f32 = jnp.float32

    def body(x_r, pr_r, win_r, bin_r, r0_r, r1_r, wa_r, wb_r, be_r,
             h_o, a_o, b_o):
        p = pr_r[...]
        h0 = (jnp.dot(x_r[...], win_r[...], preferred_element_type=f32)
              + bin_r[...] + (1.0 - p) * r0_r[...] + p * r1_r[...])
        h_o[...] = h0
        a_o[...] = _pack_halves(
            jnp.dot(h0, wa_r[...], preferred_element_type=f32) + be_r[...])
        b_o[...] = _pack_halves(
            jnp.dot(h0, wb_r[...], preferred_element_type=f32))

    return pl.pallas_call(
        body,
        out_shape=(jax.ShapeDtypeStruct((N, H), f32),
                   jax.ShapeDtypeStruct((N, H // 2), f32),
                   jax.ShapeDtypeStruct((N, H // 2), f32)),
    )(x, pring, Win, b_in, ring0, ring1, wea0, web0, be10)


def _edge0_call(ga, gb, rad2d, edge_attr, Wb1, bb1, Wb2, bb2, w1tl, w1th,
                we2l, we2h, be2, wattr, battb):
    f32 = jnp.float32

    def body(ga_r, gb_r, rad_r, ea_r, wb1_r, bb1_r, wb2_r, bb2_r, w1tl_r,
             w1th_r, we2l_r, we2h_r, be2_r, watt_r, batt_r, eat_o, m_o):
        a_lo, a_hi = _unpack_halves(ga_r[...])
        b_lo, b_hi = _unpack_halves(gb_r[...])
        radial = rad_r[...]
        d = jnp.sqrt(radial + 1e-8)
        kidx = lax.broadcasted_iota(jnp.int32, (1, DIST_DIM // 2), 1)
        freqs = (2.0 * math.pi / 15.0) * jnp.exp2(2.0 * kidx.astype(f32))
        ang = d * freqs
        bond = jnp.dot(_silu(jnp.dot(ea_r[...], wb1_r[...],
                                     preferred_element_type=f32) + bb1_r[...]),
                       wb2_r[...], preferred_element_type=f32) + bb2_r[...]
        eat = jnp.concatenate(
            [jnp.sin(ang), jnp.cos(ang), bond, jnp.zeros((EB, 4), f32)], axis=1)
        eat_o[...] = eat
        pre_lo = a_lo + b_lo + jnp.dot(eat, w1tl_r[...],
                                       preferred_element_type=f32)
        pre_hi = a_hi + b_hi + jnp.dot(eat, w1th_r[...],
                                       preferred_element_type=f32)
        q = _silu(jnp.dot(_silu(pre_lo), we2l_r[...],
                          preferred_element_type=f32)
                  + jnp.dot(_silu(pre_hi), we2h_r[...],
                            preferred_element_type=f32)
                  + be2_r[...])
        alog = (jnp.sum(q * watt_r[...], axis=1, keepdims=True)
                + batt_r[...][:, :1])
        m_o[...] = q * jax.nn.sigmoid(alog)

    grid = (E // EB,)
    return pl.pallas_call(
        body,
        grid=grid,
        in_specs=[
            pl.BlockSpec((EB, H // 2), lambda i: (i, 0)),
            pl.BlockSpec((EB, H // 2), lambda i: (i, 0)),
            pl.BlockSpec((EB, 1), lambda i: (i, 0)),
            pl.BlockSpec((EB, 16), lambda i: (i, 0)),
            pl.BlockSpec((16, 16), lambda i: (0, 0)),
            pl.BlockSpec((1, 16), lambda i: (0, 0)),
            pl.BlockSpec((16, 16), lambda i: (0, 0)),
            pl.BlockSpec((1, 16), lambda i: (0, 0)),
            pl.BlockSpec((32, H // 2), lambda i: (0, 0)),
            pl.BlockSpec((32, H // 2), lambda i: (0, 0)),
            pl.BlockSpec((H // 2, H), lambda i: (0, 0)),
            pl.BlockSpec((H // 2, H), lambda i: (0, 0)),
            pl.BlockSpec((1, H), lambda i: (0, 0)),
            pl.BlockSpec((1, H), lambda i: (0, 0)),
            pl.BlockSpec((1, H), lambda i: (0, 0)),
        ],
        out_specs=(pl.BlockSpec((EB, 32), lambda i: (i, 0)),
                   pl.BlockSpec((EB, H), lambda i: (i, 0))),
        out_shape=(jax.ShapeDtypeStruct((E, 32), f32),
                   jax.ShapeDtypeStruct((E, H), f32)),
    )(ga, gb, rad2d, edge_attr, Wb1, bb1, Wb2, bb2, w1tl, w1th, we2l, we2h,
      be2, wattr, battb)


def _edge_call(ga, gb, eat, w1tl, w1th, we2l, we2h, be2, wattr, battb):
    f32 = jnp.float32

    def body(ga_r, gb_r, ea_r, w1tl_r, w1th_r, we2l_r, we2h_r, be2_r, watt_r,
             batt_r, m_o):
        a_lo, a_hi = _unpack_halves(ga_r[...])
        b_lo, b_hi = _unpack_halves(gb_r[...])
        ea = ea_r[...]
        pre_lo = a_lo + b_lo + jnp.dot(ea, w1tl_r[...],
                                       preferred_element_type=f32)
        pre_hi = a_hi + b_hi + jnp.dot(ea, w1th_r[...],
                                       preferred_element_type=f32)
        q = _silu(jnp.dot(_silu(pre_lo), we2l_r[...],
                          preferred_element_type=f32)
                  + jnp.dot(_silu(pre_hi), we2h_r[...],
                            preferred_element_type=f32)
                  + be2_r[...])
        alog = (jnp.sum(q * watt_r[...], axis=1, keepdims=True)
                + batt_r[...][:, :1])
        m_o[...] = q * jax.nn.sigmoid(alog)

    grid = (E // EB,)
    return pl.pallas_call(
        body,
        grid=grid,
        in_specs=[
            pl.BlockSpec((EB, H // 2), lambda i: (i, 0)),
            pl.BlockSpec((EB, H // 2), lambda i: (i, 0)),
            pl.BlockSpec((EB, 32), lambda i: (i, 0)),
            pl.BlockSpec((32, H // 2), lambda i: (0, 0)),
            pl.BlockSpec((32, H // 2), lambda i: (0, 0)),
            pl.BlockSpec((H // 2, H), lambda i: (0, 0)),
            pl.BlockSpec((H // 2, H), lambda i: (0, 0)),
            pl.BlockSpec((1, H), lambda i: (0, 0)),
            pl.BlockSpec((1, H), lambda i: (0, 0)),
            pl.BlockSpec((1, H), lambda i: (0, 0)),
        ],
        out_specs=pl.BlockSpec((EB, H), lambda i: (i, 0)),
        out_shape=jax.ShapeDtypeStruct((E, H), f32),
    )(ga, gb, eat, w1tl, w1th, we2l, we2h, be2, wattr, battb)


def _node_call(h, parts, wn1a, wn1b, bn1, wn2, bn2, wea, web, be1n):
    f32 = jnp.float32

    def body(h_r, p_r, wa_r, wb_r, b1_r, w2_r, b2_r, wea_r, web_r, be_r,
             h_o, a_o, b_o):
        h0 = h_r[...]
        agg = p_r[...][:N, :]
        t = _silu(jnp.dot(h0, wa_r[...], preferred_element_type=f32)
                  + jnp.dot(agg, wb_r[...], preferred_element_type=f32)
                  + b1_r[...])
        hn = h0 + jnp.dot(t, w2_r[...], preferred_element_type=f32) + b2_r[...]
        h_o[...] = hn
        a_o[...] = _pack_halves(
            jnp.dot(hn, wea_r[...], preferred_element_type=f32) + be_r[...])
        b_o[...] = _pack_halves(
            jnp.dot(hn, web_r[...], preferred_element_type=f32))

    return pl.pallas_call(
        body,
        out_shape=(jax.ShapeDtypeStruct((N, H), f32),
                   jax.ShapeDtypeStruct((N, H // 2), f32),
                   jax.ShapeDtypeStruct((N, H // 2), f32)),
    )(h, parts, wn1a, wn1b, bn1, wn2, bn2, wea, web, be1n)


def _final_call(h, parts, wn1a, wn1b, bn1, wn2, bn2, Wo1, bo1, wo2r, bo2b,
                batch2d):
    f32 = jnp.float32

    def body(h_r, p_r, wa_r, wb_r, b1_r, w2_r, b2_r, wo1_r, bo1_r, wo2_r,
             bo2_r, bat_r, out_o):
        h0 = h_r[...]
        agg = p_r[...][:N, :]
        t = _silu(jnp.dot(h0, wa_r[...], preferred_element_type=f32)
                  + jnp.dot(agg, wb_r[...], preferred_element_type=f32)
                  + b1_r[...])
        hn = h0 + jnp.dot(t, w2_r[...], preferred_element_type=f32) + b2_r[...]
        u = jax.nn.relu(jnp.dot(hn, wo1_r[...], preferred_element_type=f32)
                        + bo1_r[...])
        logits = (jnp.sum(u * wo2_r[...], axis=1, keepdims=True)
                  + bo2_r[...][:, :1])
        gids = lax.broadcasted_iota(jnp.int32, (1, G), 1)
        mask = bat_r[...] == gids
        neg = jnp.float32(-1e30)
        cnt = jnp.sum(mask.astype(f32), axis=0, keepdims=True)
        gmax = jnp.max(jnp.where(mask, logits, neg), axis=0, keepdims=True)
        gmax = jnp.where(cnt > 0.0, gmax, 0.0)
        gmax_n = jnp.sum(jnp.where(mask, gmax, 0.0), axis=1, keepdims=True)
        ex = jnp.exp(logits - gmax_n)
        z = jnp.sum(jnp.where(mask, ex, 0.0), axis=0, keepdims=True)
        z_n = jnp.sum(jnp.where(mask, z, 0.0), axis=1, keepdims=True)
        probs = ex / (z_n + 1e-12)
        pmax = jnp.max(jnp.where(mask, probs, neg), axis=0, keepdims=True)
        pmax = jnp.where(cnt > 0.0, pmax, 0.0)
        out_o[...] = jnp.log(pmax + 1e-9)

    return pl.pallas_call(
        body,
        out_shape=jax.ShapeDtypeStruct((1, G), f32),
    )(h, parts, wn1a, wn1b, bn1, wn2, bn2, Wo1, bo1, wo2r, bo2b, batch2d)


def kernel(x, pos, edge_index, edge_attr, pring_out, batch,
           Wb1, bb1, Wb2, bb2, Win, b_in, ring_emb,
           We1, be1, We2, be2, Wn1, bn1, Wn2, bn2, Watt, batt,
           Wo1, bo1, Wo2, bo2):
    f32 = jnp.float32
    row = edge_index[0]
    col = edge_index[1]

    wea = We1[:, :H, :]
    web = We1[:, H:2 * H, :]
    w1t = jnp.pad(We1[:, 2 * H:, :], ((0, 0), (0, 4), (0, 0)))
    w1tl = w1t[:, :, :H // 2]
    w1th = w1t[:, :, H // 2:]
    we2l = We2[:, :H // 2, :]
    we2h = We2[:, H // 2:, :]
    be1r = be1.reshape(NL, 1, H)
    be2r = be2.reshape(NL, 1, H)
    bn1r = bn1.reshape(NL, 1, H)
    bn2r = bn2.reshape(NL, 1, H)
    wn1a = Wn1[:, :H, :]
    wn1b = Wn1[:, H:, :]
    wattr = Watt[:, :, 0].reshape(NL, 1, H)
    battb = jnp.broadcast_to(batt.reshape(NL, 1, 1), (NL, 1, H))
    bb1r = bb1.reshape(1, 16)
    bb2r = bb2.reshape(1, 16)
    b_inr = b_in.reshape(1, H)
    ring0 = ring_emb[0:1, :]
    ring1 = ring_emb[1:2, :]
    bo1r = bo1.reshape(1, 2 * H)
    wo2r = Wo2.reshape(1, 2 * H)
    bo2b = jnp.broadcast_to(bo2.reshape(1, 1), (1, H))
    pring = pring_out.astype(f32).reshape(N, 1)
    pos4 = jnp.pad(pos, ((0, 0), (0, 1)))
    batch2d = batch.reshape(N, 1)
    zeros_h = jnp.zeros((NPAD, H // 2), f32)

    h, atab, btab = _prep_call(x, pring, Win, b_inr, ring0, ring1,
                               wea[0], web[0], be1r[0])

    eat = None
    out = None
    for l in range(NL):
        if l == 0:
            ga, gb, rad = _gather_tables_rad(atab, btab, row, col,
                                             pos4.reshape(N * 4))
            eat, m = _edge0_call(ga, gb, rad.reshape(E, 1), edge_attr,
                                 Wb1, bb1r, Wb2, bb2r, w1tl[0], w1th[0],
                                 we2l[0], we2h[0], be2r[0], wattr[0],
                                 battb[0])
        else:
            ga, gb = _gather_tables(atab, btab, row, col)
            m = _edge_call(ga, gb, eat, w1tl[l], w1th[l], we2l[l], we2h[l],
                           be2r[l], wattr[l], battb[l])
        parts = _scatter_sum(m, row, zeros_h)
        if l < NL - 1:
            h, atab, btab = _node_call(h, parts, wn1a[l], wn1b[l], bn1r[l],
                                       Wn2[l], bn2r[l], wea[l + 1],
                                       web[l + 1], be1r[l + 1])
        else:
            out = _final_call(h, parts, wn1a[l], wn1b[l], bn1r[l], Wn2[l],
                              bn2r[l], Wo1, bo1r, wo2r, bo2b, batch2d)

    return out.reshape(G)


# speedup vs baseline: 1.4391x; 1.4016x over previous
import functools
import math

import jax
import jax.numpy as jnp
from jax import lax
from jax.experimental import pallas as pl
from jax.experimental.pallas import tpu as pltpu
from jax.experimental.pallas import tpu_sc as plsc

N = 10000
E = 320000
H = 128
G = 100
NL = 6
DIST_DIM = 12
W0 = 144
NW = 32
PER_W = E // NW
NB = 3
EB = 640
NPAD = 10240
RT = NPAD // 16

_FREQS = [2.0 * math.pi * (4.0 ** k) / 15.0 for k in range(DIST_DIM // 2)]


def _silu(v):
    return v * jax.nn.sigmoid(v)


def _unpack_halves(x):
    u = lax.bitcast_convert_type(x, jnp.uint32)
    lo = lax.bitcast_convert_type(u << 16, jnp.float32)
    hi = lax.bitcast_convert_type(u & jnp.uint32(0xFFFF0000), jnp.float32)
    return lo, hi


def _pack_halves(a):
    u16 = jnp.uint16
    u32 = jnp.uint32
    lo = lax.bitcast_convert_type(a[:, :64].astype(jnp.bfloat16), u16)
    hi = lax.bitcast_convert_type(a[:, 64:].astype(jnp.bfloat16), u16)
    packed = lo.astype(u32) | (hi.astype(u32) << 16)
    return lax.bitcast_convert_type(packed, jnp.float32)


def _make_sc_gather(with_radial):
    mesh = plsc.VectorSubcoreMesh(core_axis_name="c", subcore_axis_name="s")
    f32 = jnp.float32
    i32 = jnp.int32
    CK = 128
    NF = PER_W // CK
    NF -= NF % NB
    TL = PER_W - NF * CK

    WP = H // 2
    out_type = [jax.ShapeDtypeStruct((E, H), f32)]
    scratch = []
    for _ in range(NB):
        scratch.extend([
            pltpu.VMEM((CK,), i32), pltpu.VMEM((CK,), i32),
            pltpu.VMEM((CK, WP), f32), pltpu.VMEM((CK, WP), f32),
            pltpu.SemaphoreType.DMA, pltpu.SemaphoreType.DMA,
            pltpu.SemaphoreType.DMA,
        ])
    scratch.extend([
        pltpu.VMEM((TL,), i32), pltpu.VMEM((TL,), i32),
        pltpu.VMEM((TL, WP), f32), pltpu.VMEM((TL, WP), f32),
    ])
    if with_radial:
        out_type.append(jax.ShapeDtypeStruct((E,), f32))
        scratch.append(pltpu.VMEM((N * 4,), f32))
        for _ in range(NB):
            scratch.append(pltpu.VMEM((CK,), f32))
        scratch.append(pltpu.VMEM((TL,), f32))

    @functools.partial(
        pl.kernel,
        out_type=tuple(out_type),
        mesh=mesh,
        scratch_types=scratch,
        compiler_params=pltpu.CompilerParams(
            needs_layout_passes=False, use_tc_tiling_on_sc=False),
    )
    def gather_k(*refs):
        atab, btab, row, col = refs[:4]
        k = 4
        if with_radial:
            pos4 = refs[k]; k += 1
        oab = refs[k]; k += 1
        if with_radial:
            orad = refs[k]; k += 1
        ridx, cidx, bufa, bufb, isem, gsem, wsem = [], [], [], [], [], [], []
        for _ in range(NB):
            ridx.append(refs[k]); cidx.append(refs[k + 1])
            bufa.append(refs[k + 2]); bufb.append(refs[k + 3])
            isem.append(refs[k + 4]); gsem.append(refs[k + 5])
            wsem.append(refs[k + 6])
            k += 7
        ridxt, cidxt, bufat, bufbt = refs[k:k + 4]; k += 4
        if with_radial:
            posv = refs[k]; k += 1
            radb = refs[k:k + NB]; k += NB
            radbt = refs[k]; k += 1

        wid = lax.axis_index("s") * 2 + lax.axis_index("c")
        base = wid * PER_W
        if with_radial:
            pltpu.sync_copy(pos4, posv)

        def radial_into(rref, cref, dst, n):
            for g in range(n // 16):
                r16 = rref[pl.ds(g * 16, 16)] * 4
                c16 = cref[pl.ds(g * 16, 16)] * 4
                acc = jnp.zeros((16,), f32)
                for comp in range(3):
                    dv = (plsc.load_gather(posv, [r16 + comp])
                          - plsc.load_gather(posv, [c16 + comp]))
                    acc = acc + dv * dv
                dst[pl.ds(g * 16, 16)] = acc

        def issue_idx(c, b):
            off = base + c * CK
            pltpu.async_copy(row.at[pl.ds(off, CK)], ridx[b], isem[b])
            pltpu.async_copy(col.at[pl.ds(off, CK)], cidx[b], isem[b])

        def take_gather(c, b):
            pltpu.make_async_copy(row.at[pl.ds(base + c * CK, CK)],
                                  ridx[b], isem[b]).wait()
            pltpu.make_async_copy(col.at[pl.ds(base + c * CK, CK)],
                                  cidx[b], isem[b]).wait()
            pltpu.async_copy(atab.at[ridx[b]], bufa[b], gsem[b])
            pltpu.async_copy(btab.at[cidx[b]], bufb[b], gsem[b])
            if with_radial:
                radial_into(ridx[b], cidx[b], radb[b], CK)

        def issue_write(c, b):
            off = base + c * CK
            pltpu.make_async_copy(atab.at[ridx[b]], bufa[b],
                                  gsem[b]).wait()
            pltpu.make_async_copy(btab.at[cidx[b]], bufb[b],
                                  gsem[b]).wait()
            pltpu.async_copy(bufa[b], oab.at[pl.ds(off, CK), pl.ds(0, WP)],
                             wsem[b])
            pltpu.async_copy(bufb[b], oab.at[pl.ds(off, CK), pl.ds(WP, WP)],
                             wsem[b])
            if with_radial:
                pltpu.async_copy(radb[b], orad.at[pl.ds(off, CK)], wsem[b])

        def wait_write(c, b):
            off = base + c * CK
            pltpu.make_async_copy(bufa[b],
                                  oab.at[pl.ds(off, CK), pl.ds(0, WP)],
                                  wsem[b]).wait()
            pltpu.make_async_copy(bufb[b],
                                  oab.at[pl.ds(off, CK), pl.ds(WP, WP)],
                                  wsem[b]).wait()
            if with_radial:
                pltpu.make_async_copy(radb[b], orad.at[pl.ds(off, CK)],
                                      wsem[b]).wait()

        def body(j, carry):
            for b in range(NB):
                i = j * NB + b

                @pl.when(j >= 1)
                def _(b=b, i=i):
                    wait_write(i - NB, b)

                issue_idx(i, b)
                if b == 0:
                    @pl.when(j >= 1)
                    def _(b=b, i=i):
                        take_gather(i - 1, (b - 1) % NB)
                else:
                    take_gather(i - 1, b - 1)
                if b <= 1:
                    @pl.when(j >= 1)
                    def _(b=b, i=i):
                        issue_write(i - 2, (b - 2) % NB)
                else:
                    issue_write(i - 2, b - 2)
            return carry

        lax.fori_loop(0, NF // NB, body, 0)

        L = NF - 1
        take_gather(L, L % NB)
        issue_write(L - 1, (L - 1) % NB)
        issue_write(L, L % NB)
        wait_write(L - 2, (L - 2) % NB)
        wait_write(L - 1, (L - 1) % NB)
        wait_write(L, L % NB)

        off = base + NF * CK
        pltpu.sync_copy(row.at[pl.ds(off, TL)], ridxt)
        pltpu.sync_copy(col.at[pl.ds(off, TL)], cidxt)
        ca = pltpu.async_copy(atab.at[ridxt], bufat, gsem[0])
        cb = pltpu.async_copy(btab.at[cidxt], bufbt, gsem[1])
        if with_radial:
            radial_into(ridxt, cidxt, radbt, TL)
        ca.wait()
        cb.wait()
        pltpu.sync_copy(bufat, oab.at[pl.ds(off, TL), pl.ds(0, WP)])
        pltpu.sync_copy(bufbt, oab.at[pl.ds(off, TL), pl.ds(WP, WP)])
        if with_radial:
            pltpu.sync_copy(radbt, orad.at[pl.ds(off, TL)])

    return gather_k


_make_sc_gather = functools.lru_cache(maxsize=None)(_make_sc_gather)


def _make_sc_scatter():
    mesh = plsc.VectorSubcoreMesh(core_axis_name="c", subcore_axis_name="s")
    f32 = jnp.float32
    HC = H // 2
    SPER = E // 16
    CK = 128
    SNFULL = SPER // CK
    SNFULL -= SNFULL % NB
    STAIL = SPER - SNFULL * CK

    scratch = [pltpu.VMEM((RT, HC), f32)]
    for _ in range(NB):
        scratch.extend([
            pltpu.VMEM((CK,), jnp.int32), pltpu.VMEM((CK, HC), f32),
            pltpu.SemaphoreType.DMA, pltpu.SemaphoreType.DMA,
        ])
    scratch.extend([
        pltpu.VMEM((STAIL,), jnp.int32), pltpu.VMEM((STAIL, HC), f32),
        pltpu.VMEM_SHARED((NPAD, HC), f32),
    ])

    @functools.partial(
        pl.kernel,
        out_type=jax.ShapeDtypeStruct((NPAD, H), f32),
        mesh=mesh,
        scratch_types=scratch,
        compiler_params=pltpu.CompilerParams(
            needs_layout_passes=False, use_tc_tiling_on_sc=False),
    )
    def scatter_k(*refs):
        m, row, zeros_h, out = refs[:4]
        stage = refs[4]
        k = 5
        idxb, mbuf, lsem, ssem = [], [], [], []
        for _ in range(NB):
            idxb.append(refs[k]); mbuf.append(refs[k + 1])
            lsem.append(refs[k + 2]); ssem.append(refs[k + 3])
            k += 4
        idxt, mbuft, shared = refs[k:k + 3]

        c = lax.axis_index("c")
        s = lax.axis_index("s")
        base = s * SPER
        col0 = c * HC

        pltpu.sync_copy(zeros_h.at[pl.ds(s * RT, RT), pl.ds(0, HC)], stage)
        pltpu.sync_copy(stage, shared.at[pl.ds(s * RT, RT)])
        plsc.subcore_barrier()

        def issue_load(i, b):
            off = base + i * CK
            pltpu.async_copy(row.at[pl.ds(off, CK)], idxb[b], lsem[b])
            pltpu.async_copy(m.at[pl.ds(off, CK), pl.ds(col0, HC)],
                             mbuf[b], lsem[b])

        def take_scatter(i, b):
            off = base + i * CK
            pltpu.make_async_copy(row.at[pl.ds(off, CK)], idxb[b],
                                  lsem[b]).wait()
            pltpu.make_async_copy(m.at[pl.ds(off, CK), pl.ds(col0, HC)],
                                  mbuf[b], lsem[b]).wait()
            pltpu.async_copy(mbuf[b], shared.at[idxb[b]], ssem[b], add=True)

        def wait_scatter(b):
            pltpu.make_async_copy(mbuf[b], shared.at[idxb[b]],
                                  ssem[b]).wait()

        def body(j, carry):
            for b in range(NB):
                i = j * NB + b

                @pl.when(j >= 1)
                def _(b=b):
                    wait_scatter(b)

                issue_load(i, b)
                if b == 0:
                    @pl.when(j >= 1)
                    def _(b=b, i=i):
                        take_scatter(i - 1, (b - 1) % NB)
                else:
                    take_scatter(i - 1, b - 1)
            return carry

        lax.fori_loop(0, SNFULL // NB, body, 0)

        L = SNFULL - 1
        take_scatter(L, L % NB)
        for b in range(NB):
            wait_scatter(b)

        if STAIL:
            off = base + SNFULL * CK
            pltpu.sync_copy(row.at[pl.ds(off, STAIL)], idxt)
            pltpu.sync_copy(m.at[pl.ds(off, STAIL), pl.ds(col0, HC)], mbuft)
            pltpu.sync_copy(mbuft, shared.at[idxt], add=True)

        plsc.subcore_barrier()
        pltpu.sync_copy(shared.at[pl.ds(s * RT, RT)], stage)
        pltpu.sync_copy(stage, out.at[pl.ds(s * RT, RT), pl.ds(col0, HC)])

    return scatter_k


_make_sc_scatter = functools.lru_cache(maxsize=None)(_make_sc_scatter)


def _gather_tables(atab, btab, row, col):
    res = _make_sc_gather(False)(atab, btab, row, col)
    return res[0] if isinstance(res, (tuple, list)) else res


def _gather_tables_rad(atab, btab, row, col, pos4):
    return _make_sc_gather(True)(atab, btab, row, col, pos4)


def _scatter_sum(m, row, zeros_h):
    return _make_sc_scatter()(m, row, zeros_h)


def _full(shape):
    return pl.BlockSpec(shape, lambda: tuple(0 for _ in shape))


def _prep_call(x, pring, Win, b_in, ring0, ring1, wea0, web0, be10):
    f32 = jnp.float32

    def body(x_r, pr_r, win_r, bin_r, r0_r, r1_r, wa_r, wb_r, be_r,
             h_o, a_o, b_o):
        p = pr_r[...]
        h0 = (jnp.dot(x_r[...], win_r[...], preferred_element_type=f32)
              + bin_r[...] + (1.0 - p) * r0_r[...] + p * r1_r[...])
        h_o[...] = h0
        a_o[...] = _pack_halves(
            jnp.dot(h0, wa_r[...], preferred_element_type=f32) + be_r[...])
        b_o[...] = _pack_halves(
            jnp.dot(h0, wb_r[...], preferred_element_type=f32))

    return pl.pallas_call(
        body,
        out_shape=(jax.ShapeDtypeStruct((N, H), f32),
                   jax.ShapeDtypeStruct((N, H // 2), f32),
                   jax.ShapeDtypeStruct((N, H // 2), f32)),
    )(x, pring, Win, b_in, ring0, ring1, wea0, web0, be10)


def _edge0_call(gab, rad2d, edge_attr, Wb1, bb1, Wb2, bb2, w1tl, w1th,
                we2l, we2h, be2, wattr, battb):
    f32 = jnp.float32

    def body(gab_r, rad_r, ea_r, wb1_r, bb1_r, wb2_r, bb2_r, w1tl_r,
             w1th_r, we2l_r, we2h_r, be2_r, watt_r, batt_r, eat_o, m_o):
        g = gab_r[...]
        a_lo, a_hi = _unpack_halves(g[:, :H // 2])
        b_lo, b_hi = _unpack_halves(g[:, H // 2:])
        radial = rad_r[...]
        d = jnp.sqrt(radial + 1e-8)
        kidx = lax.broadcasted_iota(jnp.int32, (1, DIST_DIM // 2), 1)
        freqs = (2.0 * math.pi / 15.0) * jnp.exp2(2.0 * kidx.astype(f32))
        ang = d * freqs
        bond = jnp.dot(_silu(jnp.dot(ea_r[...], wb1_r[...],
                                     preferred_element_type=f32) + bb1_r[...]),
                       wb2_r[...], preferred_element_type=f32) + bb2_r[...]
        eat = jnp.concatenate(
            [jnp.sin(ang), jnp.cos(ang), bond, jnp.zeros((EB, 4), f32)], axis=1)
        eat_o[...] = eat
        pre_lo = a_lo + b_lo + jnp.dot(eat, w1tl_r[...],
                                       preferred_element_type=f32)
        pre_hi = a_hi + b_hi + jnp.dot(eat, w1th_r[...],
                                       preferred_element_type=f32)
        q = _silu(jnp.dot(_silu(pre_lo), we2l_r[...],
                          preferred_element_type=f32)
                  + jnp.dot(_silu(pre_hi), we2h_r[...],
                            preferred_element_type=f32)
                  + be2_r[...])
        alog = (jnp.sum(q * watt_r[...], axis=1, keepdims=True)
                + batt_r[...][:, :1])
        m_o[...] = q * jax.nn.sigmoid(alog)

    grid = (E // EB,)
    return pl.pallas_call(
        body,
        grid=grid,
        in_specs=[
            pl.BlockSpec((EB, H), lambda i: (i, 0)),
            pl.BlockSpec((EB, 1), lambda i: (i, 0)),
            pl.BlockSpec((EB, 16), lambda i: (i, 0)),
            pl.BlockSpec((16, 16), lambda i: (0, 0)),
            pl.BlockSpec((1, 16), lambda i: (0, 0)),
            pl.BlockSpec((16, 16), lambda i: (0, 0)),
            pl.BlockSpec((1, 16), lambda i: (0, 0)),
            pl.BlockSpec((32, H // 2), lambda i: (0, 0)),
            pl.BlockSpec((32, H // 2), lambda i: (0, 0)),
            pl.BlockSpec((H // 2, H), lambda i: (0, 0)),
            pl.BlockSpec((H // 2, H), lambda i: (0, 0)),
            pl.BlockSpec((1, H), lambda i: (0, 0)),
            pl.BlockSpec((1, H), lambda i: (0, 0)),
            pl.BlockSpec((1, H), lambda i: (0, 0)),
        ],
        out_specs=(pl.BlockSpec((EB, 32), lambda i: (i, 0)),
                   pl.BlockSpec((EB, H), lambda i: (i, 0))),
        out_shape=(jax.ShapeDtypeStruct((E, 32), f32),
                   jax.ShapeDtypeStruct((E, H), f32)),
    )(gab, rad2d, edge_attr, Wb1, bb1, Wb2, bb2, w1tl, w1th, we2l, we2h,
      be2, wattr, battb)


def _edge_call(gab, eat, w1tl, w1th, we2l, we2h, be2, wattr, battb):
    f32 = jnp.float32

    def body(gab_r, ea_r, w1tl_r, w1th_r, we2l_r, we2h_r, be2_r, watt_r,
             batt_r, m_o):
        g = gab_r[...]
        a_lo, a_hi = _unpack_halves(g[:, :H // 2])
        b_lo, b_hi = _unpack_halves(g[:, H // 2:])
        ea = ea_r[...]
        pre_lo = a_lo + b_lo + jnp.dot(ea, w1tl_r[...],
                                       preferred_element_type=f32)
        pre_hi = a_hi + b_hi + jnp.dot(ea, w1th_r[...],
                                       preferred_element_type=f32)
        q = _silu(jnp.dot(_silu(pre_lo), we2l_r[...],
                          preferred_element_type=f32)
                  + jnp.dot(_silu(pre_hi), we2h_r[...],
                            preferred_element_type=f32)
                  + be2_r[...])
        alog = (jnp.sum(q * watt_r[...], axis=1, keepdims=True)
                + batt_r[...][:, :1])
        m_o[...] = q * jax.nn.sigmoid(alog)

    grid = (E // EB,)
    return pl.pallas_call(
        body,
        grid=grid,
        in_specs=[
            pl.BlockSpec((EB, H), lambda i: (i, 0)),
            pl.BlockSpec((EB, 32), lambda i: (i, 0)),
            pl.BlockSpec((32, H // 2), lambda i: (0, 0)),
            pl.BlockSpec((32, H // 2), lambda i: (0, 0)),
            pl.BlockSpec((H // 2, H), lambda i: (0, 0)),
            pl.BlockSpec((H // 2, H), lambda i: (0, 0)),
            pl.BlockSpec((1, H), lambda i: (0, 0)),
            pl.BlockSpec((1, H), lambda i: (0, 0)),
            pl.BlockSpec((1, H), lambda i: (0, 0)),
        ],
        out_specs=pl.BlockSpec((EB, H), lambda i: (i, 0)),
        out_shape=jax.ShapeDtypeStruct((E, H), f32),
    )(gab, eat, w1tl, w1th, we2l, we2h, be2, wattr, battb)


def _node_call(h, parts, wn1a, wn1b, bn1, wn2, bn2, wea, web, be1n):
    f32 = jnp.float32

    def body(h_r, p_r, wa_r, wb_r, b1_r, w2_r, b2_r, wea_r, web_r, be_r,
             h_o, a_o, b_o):
        h0 = h_r[...]
        agg = p_r[...][:N, :]
        t = _silu(jnp.dot(h0, wa_r[...], preferred_element_type=f32)
                  + jnp.dot(agg, wb_r[...], preferred_element_type=f32)
                  + b1_r[...])
        hn = h0 + jnp.dot(t, w2_r[...], preferred_element_type=f32) + b2_r[...]
        h_o[...] = hn
        a_o[...] = _pack_halves(
            jnp.dot(hn, wea_r[...], preferred_element_type=f32) + be_r[...])
        b_o[...] = _pack_halves(
            jnp.dot(hn, web_r[...], preferred_element_type=f32))

    return pl.pallas_call(
        body,
        out_shape=(jax.ShapeDtypeStruct((N, H), f32),
                   jax.ShapeDtypeStruct((N, H // 2), f32),
                   jax.ShapeDtypeStruct((N, H // 2), f32)),
    )(h, parts, wn1a, wn1b, bn1, wn2, bn2, wea, web, be1n)


def _final_call(h, parts, wn1a, wn1b, bn1, wn2, bn2, Wo1, bo1, wo2r, bo2b,
                batch2d):
    f32 = jnp.float32

    def body(h_r, p_r, wa_r, wb_r, b1_r, w2_r, b2_r, wo1_r, bo1_r, wo2_r,
             bo2_r, bat_r, out_o):
        h0 = h_r[...]
        agg = p_r[...][:N, :]
        t = _silu(jnp.dot(h0, wa_r[...], preferred_element_type=f32)
                  + jnp.dot(agg, wb_r[...], preferred_element_type=f32)
                  + b1_r[...])
        hn = h0 + jnp.dot(t, w2_r[...], preferred_element_type=f32) + b2_r[...]
        u = jax.nn.relu(jnp.dot(hn, wo1_r[...], preferred_element_type=f32)
                        + bo1_r[...])
        logits = (jnp.sum(u * wo2_r[...], axis=1, keepdims=True)
                  + bo2_r[...][:, :1])
        gids = lax.broadcasted_iota(jnp.int32, (1, G), 1)
        mask = bat_r[...] == gids
        neg = jnp.float32(-1e30)
        cnt = jnp.sum(mask.astype(f32), axis=0, keepdims=True)
        gmax = jnp.max(jnp.where(mask, logits, neg), axis=0, keepdims=True)
        gmax = jnp.where(cnt > 0.0, gmax, 0.0)
        gmax_n = jnp.sum(jnp.where(mask, gmax, 0.0), axis=1, keepdims=True)
        ex = jnp.exp(logits - gmax_n)
        z = jnp.sum(jnp.where(mask, ex, 0.0), axis=0, keepdims=True)
        z_n = jnp.sum(jnp.where(mask, z, 0.0), axis=1, keepdims=True)
        probs = ex / (z_n + 1e-12)
        pmax = jnp.max(jnp.where(mask, probs, neg), axis=0, keepdims=True)
        pmax = jnp.where(cnt > 0.0, pmax, 0.0)
        out_o[...] = jnp.log(pmax + 1e-9)

    return pl.pallas_call(
        body,
        out_shape=jax.ShapeDtypeStruct((1, G), f32),
    )(h, parts, wn1a, wn1b, bn1, wn2, bn2, Wo1, bo1, wo2r, bo2b, batch2d)


def kernel(x, pos, edge_index, edge_attr, pring_out, batch,
           Wb1, bb1, Wb2, bb2, Win, b_in, ring_emb,
           We1, be1, We2, be2, Wn1, bn1, Wn2, bn2, Watt, batt,
           Wo1, bo1, Wo2, bo2):
    f32 = jnp.float32
    row = edge_index[0]
    col = edge_index[1]

    wea = We1[:, :H, :]
    web = We1[:, H:2 * H, :]
    w1t = jnp.pad(We1[:, 2 * H:, :], ((0, 0), (0, 4), (0, 0)))
    w1tl = w1t[:, :, :H // 2]
    w1th = w1t[:, :, H // 2:]
    we2l = We2[:, :H // 2, :]
    we2h = We2[:, H // 2:, :]
    be1r = be1.reshape(NL, 1, H)
    be2r = be2.reshape(NL, 1, H)
    bn1r = bn1.reshape(NL, 1, H)
    bn2r = bn2.reshape(NL, 1, H)
    wn1a = Wn1[:, :H, :]
    wn1b = Wn1[:, H:, :]
    wattr = Watt[:, :, 0].reshape(NL, 1, H)
    battb = jnp.broadcast_to(batt.reshape(NL, 1, 1), (NL, 1, H))
    bb1r = bb1.reshape(1, 16)
    bb2r = bb2.reshape(1, 16)
    b_inr = b_in.reshape(1, H)
    ring0 = ring_emb[0:1, :]
    ring1 = ring_emb[1:2, :]
    bo1r = bo1.reshape(1, 2 * H)
    wo2r = Wo2.reshape(1, 2 * H)
    bo2b = jnp.broadcast_to(bo2.reshape(1, 1), (1, H))
    pring = pring_out.astype(f32).reshape(N, 1)
    pos4 = jnp.pad(pos, ((0, 0), (0, 1)))
    batch2d = batch.reshape(N, 1)
    zeros_h = jnp.zeros((NPAD, H), f32)

    h, atab, btab = _prep_call(x, pring, Win, b_inr, ring0, ring1,
                               wea[0], web[0], be1r[0])

    eat = None
    out = None
    for l in range(NL):
        if l == 0:
            gab, rad = _gather_tables_rad(atab, btab, row, col,
                                          pos4.reshape(N * 4))
            eat, m = _edge0_call(gab, rad.reshape(E, 1), edge_attr,
                                 Wb1, bb1r, Wb2, bb2r, w1tl[0], w1th[0],
                                 we2l[0], we2h[0], be2r[0], wattr[0],
                                 battb[0])
        else:
            gab = _gather_tables(atab, btab, row, col)
            m = _edge_call(gab, eat, w1tl[l], w1th[l], we2l[l], we2h[l],
                           be2r[l], wattr[l], battb[l])
        parts = _scatter_sum(m, row, zeros_h)
        if l < NL - 1:
            h, atab, btab = _node_call(h, parts, wn1a[l], wn1b[l], bn1r[l],
                                       Wn2[l], bn2r[l], wea[l + 1],
                                       web[l + 1], be1r[l + 1])
        else:
            out = _final_call(h, parts, wn1a[l], wn1b[l], bn1r[l], Wn2[l],
                              bn2r[l], Wo1, bo1r, wo2r, bo2b, batch2d)

    return out.reshape(G)
